# Initial kernel scaffold; baseline (speedup 1.0000x reference)
import jax
import jax.numpy as jnp
from jax.experimental import pallas as pl


def kernel(flat, segment_ids, W, b):
    raise NotImplementedError("write your pallas kernel here")


# speedup vs baseline: 1.4404x; 1.4404x over previous
import functools

import jax
import jax.numpy as jnp
from jax import lax
from jax.experimental import pallas as pl
from jax.experimental.pallas import tpu as pltpu
from jax.experimental.pallas import tpu_sc as plsc

BATCH = 16
TOTAL_TOKENS = 32768
D_IN = 512
D_OUT = 64

NC = 2
NS = 16
NW = NC * NS
ROWS_PER_W = TOTAL_TOKENS // NW
CHUNK = 64
NCH = ROWS_PER_W // CHUNK

_sc_mesh = plsc.VectorSubcoreMesh(
    core_axis_name="c", subcore_axis_name="s", num_cores=NC, num_subcores=NS
)


@functools.partial(
    pl.kernel,
    out_type=(
        jax.ShapeDtypeStruct((NW, BATCH, D_IN), jnp.float32),
        jax.ShapeDtypeStruct((NW, BATCH, 16), jnp.float32),
    ),
    mesh=_sc_mesh,
    scratch_types=[
        pltpu.VMEM((NCH, CHUNK), jnp.int32),
        pltpu.VMEM((CHUNK, D_IN), jnp.float32),
        pltpu.VMEM((BATCH, D_IN), jnp.float32),
        pltpu.VMEM((BATCH, 16), jnp.float32),
    ],
)
def _sc_segment_sums(flat_hbm, seg_hbm, out_sum, out_cnt,
                     idx_v, buf, acc, accc):
    cid = lax.axis_index("c")
    sid = lax.axis_index("s")
    wid = sid * NC + cid

    pltpu.sync_copy(seg_hbm.at[wid], idx_v)

    zero = jnp.zeros((16,), jnp.float32)
    one = jnp.ones((16,), jnp.float32)
    for i in range(BATCH):
        for j in range(D_IN // 16):
            acc[i, pl.ds(j * 16, 16)] = zero
        accc[i, :] = zero

    base = wid * ROWS_PER_W

    def _chunk(j, _):
        pltpu.sync_copy(flat_hbm.at[pl.ds(base + j * CHUNK, CHUNK)], buf)
        for g in range(CHUNK // 16):
            seg16 = idx_v[j, pl.ds(g * 16, 16)]
            for t in range(16):
                s = seg16[t]
                r = g * 16 + t
                for jb in range(D_IN // 16):
                    plsc.addupdate(acc.at[s, pl.ds(jb * 16, 16)],
                                   buf[r, pl.ds(jb * 16, 16)])
                plsc.addupdate(accc.at[s, :], one)
        return 0

    lax.fori_loop(0, NCH, _chunk, 0)

    pltpu.sync_copy(acc, out_sum.at[wid])
    pltpu.sync_copy(accc, out_cnt.at[wid])


def _tc_finish(psum_ref, pcnt_ref, w_ref, b_ref, o_ref):
    sums = jnp.sum(psum_ref[...], axis=0)
    cnts = jnp.sum(pcnt_ref[...], axis=0)
    cnt = cnts[:, 0:1]
    avg = sums / jnp.maximum(cnt, 1.0)
    o_ref[...] = lax.dot_general(
        avg, w_ref[...], (((1,), (1,)), ((), ())),
        preferred_element_type=jnp.float32,
    ) + b_ref[...]


def kernel(flat, segment_ids, W, b):
    seg3 = segment_ids.astype(jnp.int32).reshape(NW, NCH, CHUNK)
    psum, pcnt = _sc_segment_sums(flat, seg3)
    out = pl.pallas_call(
        _tc_finish,
        out_shape=jax.ShapeDtypeStruct((BATCH, D_OUT), jnp.float32),
    )(psum, pcnt, W, b.reshape(1, D_OUT))
    return out


# speedup vs baseline: 2.9424x; 2.0428x over previous
import functools

import jax
import jax.numpy as jnp
from jax import lax
from jax.experimental import pallas as pl
from jax.experimental.pallas import tpu as pltpu
from jax.experimental.pallas import tpu_sc as plsc

BATCH = 16
TOTAL_TOKENS = 32768
D_IN = 512
D_OUT = 64

NC = 2
NS = 16
NW = NC * NS
ROWS_PER_W = TOTAL_TOKENS // NW
CHUNK = 512
NCH = ROWS_PER_W // CHUNK

MM_BLK = 512


def _tc_matmul(flat_ref, w_ref, y_ref):
    y_ref[...] = lax.dot_general(
        flat_ref[...], w_ref[...], (((1,), (1,)), ((), ())),
        preferred_element_type=jnp.float32,
    )


_sc_mesh = plsc.VectorSubcoreMesh(
    core_axis_name="c", subcore_axis_name="s", num_cores=NC, num_subcores=NS
)


@functools.partial(
    pl.kernel,
    out_type=(
        jax.ShapeDtypeStruct((NW, BATCH, D_OUT), jnp.float32),
        jax.ShapeDtypeStruct((NW, BATCH, 16), jnp.float32),
    ),
    mesh=_sc_mesh,
    scratch_types=[
        pltpu.VMEM((ROWS_PER_W // 16, 16), jnp.int32),
        pltpu.VMEM((CHUNK, D_OUT), jnp.float32),
        pltpu.VMEM((BATCH, D_OUT), jnp.float32),
        pltpu.VMEM((BATCH, 16), jnp.float32),
    ],
)
def _sc_segment_sums(y_hbm, seg_hbm, out_sum, out_cnt, idx_v, buf, acc, accc):
    cid = lax.axis_index("c")
    sid = lax.axis_index("s")
    wid = sid * NC + cid

    pltpu.sync_copy(seg_hbm.at[wid], idx_v)

    zero = jnp.zeros((16,), jnp.float32)
    for i in range(BATCH):
        for j in range(D_OUT // 16):
            acc[i, pl.ds(j * 16, 16)] = zero
        accc[i, :] = zero

    def _hist(g, _):
        ids = idx_v[g, :]
        for s in range(BATCH):
            sel = jnp.where(ids == s, 1.0, 0.0)
            plsc.addupdate(accc.at[s, :], sel)
        return 0

    lax.fori_loop(0, ROWS_PER_W // 16, _hist, 0)

    base = wid * ROWS_PER_W

    def _chunk(j, _):
        pltpu.sync_copy(y_hbm.at[pl.ds(base + j * CHUNK, CHUNK)], buf)

        def _grp(g, _):
            seg16 = idx_v[j * (CHUNK // 16) + g, :]
            for t in range(16):
                s = seg16[t]
                r = g * 16 + t
                for jb in range(D_OUT // 16):
                    plsc.addupdate(acc.at[s, pl.ds(jb * 16, 16)],
                                   buf[r, pl.ds(jb * 16, 16)])
            return 0

        lax.fori_loop(0, CHUNK // 16, _grp, 0)
        return 0

    lax.fori_loop(0, NCH, _chunk, 0)

    pltpu.sync_copy(acc, out_sum.at[wid])
    pltpu.sync_copy(accc, out_cnt.at[wid])


def _tc_finish(psum_ref, pcnt_ref, b_ref, o_ref):
    sums = jnp.sum(psum_ref[...], axis=0)
    cnts = jnp.sum(pcnt_ref[...], axis=0)
    cnt = jnp.sum(cnts, axis=1, keepdims=True)
    avg = sums / jnp.maximum(cnt, 1.0)
    o_ref[...] = avg + b_ref[...]


def kernel(flat, segment_ids, W, b):
    seg3 = segment_ids.astype(jnp.int32).reshape(NW, ROWS_PER_W // 16, 16)
    y = pl.pallas_call(
        _tc_matmul,
        grid=(TOTAL_TOKENS // MM_BLK,),
        in_specs=[
            pl.BlockSpec((MM_BLK, D_IN), lambda i: (i, 0)),
            pl.BlockSpec((D_OUT, D_IN), lambda i: (0, 0)),
        ],
        out_specs=pl.BlockSpec((MM_BLK, D_OUT), lambda i: (i, 0)),
        out_shape=jax.ShapeDtypeStruct((TOTAL_TOKENS, D_OUT), jnp.float32),
    )(flat, W)
    psum, pcnt = _sc_segment_sums(y, seg3)
    out = pl.pallas_call(
        _tc_finish,
        out_shape=jax.ShapeDtypeStruct((BATCH, D_OUT), jnp.float32),
    )(psum, pcnt, b.reshape(1, D_OUT))
    return out


# speedup vs baseline: 4.1515x; 1.4109x over previous
import functools

import jax
import jax.numpy as jnp
from jax import lax
from jax.experimental import pallas as pl
from jax.experimental.pallas import tpu as pltpu
from jax.experimental.pallas import tpu_sc as plsc

BATCH = 16
TOTAL_TOKENS = 32768
D_IN = 512
D_OUT = 64

NC = 2
NS = 16
NW = NC * NS
ROWS_PER_W = TOTAL_TOKENS // NW
CHUNK = 512
NCH = ROWS_PER_W // CHUNK

MM_BLK = 2048


def _tc_matmul(flat_ref, w_ref, y_ref):
    y_ref[...] = lax.dot_general(
        flat_ref[...], w_ref[...], (((1,), (1,)), ((), ())),
        preferred_element_type=jnp.float32,
    )


_sc_mesh = plsc.VectorSubcoreMesh(
    core_axis_name="c", subcore_axis_name="s", num_cores=NC, num_subcores=NS
)


@functools.partial(
    pl.kernel,
    out_type=(
        jax.ShapeDtypeStruct((NW, BATCH, D_OUT), jnp.float32),
        jax.ShapeDtypeStruct((NW, BATCH, 16), jnp.float32),
    ),
    mesh=_sc_mesh,
    scratch_types=[
        pltpu.VMEM((ROWS_PER_W,), jnp.int32),
        pltpu.VMEM((CHUNK, D_OUT), jnp.float32),
        pltpu.VMEM((BATCH, D_OUT), jnp.float32),
        pltpu.VMEM((BATCH, 16), jnp.float32),
    ],
)
def _sc_segment_sums(y_hbm, seg_hbm, out_sum, out_cnt, idx_v, buf, acc, accc):
    cid = lax.axis_index("c")
    sid = lax.axis_index("s")
    wid = sid * NC + cid

    pltpu.sync_copy(seg_hbm.at[pl.ds(wid * ROWS_PER_W, ROWS_PER_W)], idx_v)

    zero = jnp.zeros((16,), jnp.float32)
    for i in range(BATCH):
        for j in range(D_OUT // 16):
            acc[i, pl.ds(j * 16, 16)] = zero
        accc[i, :] = zero

    def _hist(g, _):
        ids = idx_v[pl.ds(g * 16, 16)]
        for s in range(BATCH):
            sel = jnp.where(ids == s, 1.0, 0.0)
            plsc.addupdate(accc.at[s, :], sel)
        return 0

    lax.fori_loop(0, ROWS_PER_W // 16, _hist, 0)

    base = wid * ROWS_PER_W

    def _chunk(j, _):
        pltpu.sync_copy(y_hbm.at[pl.ds(base + j * CHUNK, CHUNK)], buf)

        def _grp(g, _):
            seg16 = idx_v[pl.ds((j * (CHUNK // 16) + g) * 16, 16)]
            for t in range(16):
                s = seg16[t]
                r = g * 16 + t
                for jb in range(D_OUT // 16):
                    plsc.addupdate(acc.at[s, pl.ds(jb * 16, 16)],
                                   buf[r, pl.ds(jb * 16, 16)])
            return 0

        lax.fori_loop(0, CHUNK // 16, _grp, 0)
        return 0

    lax.fori_loop(0, NCH, _chunk, 0)

    pltpu.sync_copy(acc, out_sum.at[wid])
    pltpu.sync_copy(accc, out_cnt.at[wid])


def _tc_finish(psum_ref, pcnt_ref, b_ref, o_ref):
    sums = jnp.sum(psum_ref[...], axis=0)
    cnts = jnp.sum(pcnt_ref[...], axis=0)
    cnt = jnp.sum(cnts, axis=1, keepdims=True)
    avg = sums / jnp.maximum(cnt, 1.0)
    o_ref[...] = avg + b_ref[...]


def kernel(flat, segment_ids, W, b):
    seg = segment_ids.astype(jnp.int32)
    y = pl.pallas_call(
        _tc_matmul,
        grid=(TOTAL_TOKENS // MM_BLK,),
        in_specs=[
            pl.BlockSpec((MM_BLK, D_IN), lambda i: (i, 0)),
            pl.BlockSpec((D_OUT, D_IN), lambda i: (0, 0)),
        ],
        out_specs=pl.BlockSpec((MM_BLK, D_OUT), lambda i: (i, 0)),
        out_shape=jax.ShapeDtypeStruct((TOTAL_TOKENS, D_OUT), jnp.float32),
    )(flat, W)
    psum, pcnt = _sc_segment_sums(y, seg)
    out = pl.pallas_call(
        _tc_finish,
        out_shape=jax.ShapeDtypeStruct((BATCH, D_OUT), jnp.float32),
    )(psum, pcnt, b.reshape(1, D_OUT))
    return out
